# Initial kernel scaffold; baseline (speedup 1.0000x reference)
import jax
import jax.numpy as jnp
from jax.experimental import pallas as pl


def kernel(feature, edge_index, edge_subgraph_idx, norm, W):
    raise NotImplementedError("write your pallas kernel here")


# speedup vs baseline: 32.2930x; 32.2930x over previous
import jax
import jax.numpy as jnp
from jax import lax
from jax.experimental import pallas as pl
from jax.experimental.pallas import tpu as pltpu
from jax.experimental.pallas import tpu_sc as plsc

N = 10000
E = 320000
D = 9
F = 128
NPAD = 10240
NC = 2
NS = 16
EA = E // NS
EC = EA // NC
NH = NPAD // NC
BR = 128
ERWS = (E // NS + BR - 1) // BR
BC = 40
NB_C = EC // BC
CH = 800
NCH = EA // CH
VR = 16
NSL = NPAD // NS
CSL = NH // NS


def _zero_f32_2d(ref, nrows, ncols):
    z = jnp.zeros((VR,), jnp.float32)

    def body(r, _):
        for k in range(ncols // VR):
            ref[r, pl.ds(k * VR, VR)] = z
        return 0

    lax.fori_loop(0, nrows, body, 0)


def _sc_body(xt_hbm, src_hbm, dst_hbm, div_hbm, s_out, c_out,
             cnt_sh, nbits_sh, acc_sh,
             ib2d, nb_l, rows, idx_sm, plane1, nb_slice, srcc, dstc,
             divc, ones_b, idx_g, idx_s, idx16a, idx16b, sem):
    c = lax.axis_index("c")
    s = lax.axis_index("s")
    base_a = pl.multiple_of(s * EA, EA)

    def zv(i, _):
        plane1[pl.ds(i * VR, VR)] = jnp.zeros((VR,), jnp.float32)
        return 0
    lax.fori_loop(0, NSL // VR, zv, 0)

    def ov(i, _):
        ones_b[pl.ds(i * VR, VR)] = jnp.ones((VR,), jnp.float32)
        return 0
    lax.fori_loop(0, BR // VR, ov, 0)
    _zero_f32_2d(rows, BC, F)

    @pl.when(s < D)
    def _():
        plane = pl.multiple_of(s * NPAD, NPAD)
        for q in range(NPAD // NSL):
            pltpu.sync_copy(plane1, cnt_sh.at[pl.ds(plane + q * NSL, NSL)])
    row0 = pl.multiple_of(s * NSL, NSL)
    for q in range(NSL // BC):
        pltpu.sync_copy(rows, acc_sh.at[pl.ds(row0 + q * BC, BC)])
    plsc.subcore_barrier()

    for g in range(EA // VR, ERWS * (BR // VR)):
        ib2d[g // (BR // VR), pl.ds((g % (BR // VR)) * VR, VR)] = jnp.full(
            (VR,), D * NPAD - 1, jnp.int32)
    for ep in (0, 1):
        def build(t, _):
            pltpu.sync_copy(div_hbm.at[pl.ds(base_a + t * CH, CH)], divc)
            if ep == 0:
                pltpu.sync_copy(src_hbm.at[pl.ds(base_a + t * CH, CH)], srcc)
            else:
                pltpu.sync_copy(dst_hbm.at[pl.ds(base_a + t * CH, CH)], srcc)

            def bvec(u, _):
                g = t * (CH // VR) + u
                dv = divc[pl.ds(u * VR, VR)]
                nv = srcc[pl.ds(u * VR, VR)]
                ib2d[g // (BR // VR), pl.ds((g % (BR // VR)) * VR, VR)] = (
                    dv * NPAD + nv)
                return 0
            lax.fori_loop(0, CH // VR, bvec, 0)
            return 0
        lax.fori_loop(0, NCH, build, 0)

        KF = 8

        def scat(r, _):
            slot = r % KF

            @pl.when(r >= KF)
            def _():
                pltpu.make_async_copy(ones_b, cnt_sh.at[idx_sm.at[0]],
                                      sem).wait()
            for k in range(BR // VR):
                idx_sm[slot, pl.ds(k * VR, VR)] = ib2d[r, pl.ds(k * VR, VR)]
            pltpu.async_copy(ones_b, cnt_sh.at[idx_sm.at[slot]], sem, add=True)
            return 0
        lax.fori_loop(0, ERWS, scat, 0)

        def drain(_q, _):
            pltpu.make_async_copy(ones_b, cnt_sh.at[idx_sm.at[0]], sem).wait()
            return 0
        lax.fori_loop(0, KF, drain, 0)
    plsc.subcore_barrier()

    def zb(j, _):
        nb_slice[pl.ds(j * VR, VR)] = jnp.zeros((VR,), jnp.int32)
        return 0
    lax.fori_loop(0, NSL // VR, zb, 0)
    for i in range(D):
        pltpu.sync_copy(cnt_sh.at[pl.ds(i * NPAD + row0, NSL)], plane1)

        def orbit(j, _):
            v = plane1[pl.ds(j * VR, VR)]
            cur = nb_slice[pl.ds(j * VR, VR)]
            nb_slice[pl.ds(j * VR, VR)] = cur | jnp.where(
                v > 0.0, jnp.int32(1 << i), jnp.int32(0))
            return 0
        lax.fori_loop(0, NSL // VR, orbit, 0)
    pltpu.sync_copy(nb_slice, nbits_sh.at[pl.ds(row0, NSL)])
    plsc.subcore_barrier()

    pltpu.sync_copy(nbits_sh, nb_l)

    def mk_chunk(t, _):
        pltpu.sync_copy(src_hbm.at[pl.ds(base_a + t * CH, CH)], srcc)
        pltpu.sync_copy(dst_hbm.at[pl.ds(base_a + t * CH, CH)], dstc)

        def mk_bits(u, _):
            j = t * (CH // VR) + u
            sv = srcc[pl.ds(u * VR, VR)]
            dv = dstc[pl.ds(u * VR, VR)]
            b = plsc.load_gather(nb_l, [sv]) & plsc.load_gather(nb_l, [dv])
            ib2d[j // (BR // VR), pl.ds((j % (BR // VR)) * VR, VR)] = b
            return 0
        lax.fori_loop(0, CH // VR, mk_bits, 0)
        return 0
    lax.fori_loop(0, NCH, mk_chunk, 0)

    cbase = c * EC

    def sall(b, _):
        lo = cbase + b * BC
        pltpu.sync_copy(src_hbm.at[pl.ds(base_a + lo, BC)], idx_g)
        pltpu.sync_copy(dst_hbm.at[pl.ds(base_a + lo, BC)], idx_s)
        pltpu.sync_copy(xt_hbm.at[idx_g], rows)
        pltpu.sync_copy(rows, acc_sh.at[idx_s], add=True)
        return 0
    lax.fori_loop(0, NB_C, sall, 0)
    plsc.subcore_barrier()

    pltpu.sync_copy(acc_sh.at[pl.ds(row0, NSL)], s_out.at[c, pl.ds(row0, NSL)])
    plsc.subcore_barrier()

    _zero_f32_2d(rows, BC, F)
    lane = lax.iota(jnp.int32, VR)

    def corr(i, _):
        crow0 = pl.multiple_of(s * CSL, CSL)
        for q in range(CSL // BC):
            pltpu.sync_copy(rows, acc_sh.at[pl.ds(crow0 + q * BC, BC)])

        @pl.when(s == 0)
        def _():
            pltpu.sync_copy(rows.at[pl.ds(0, 8)], acc_sh.at[pl.ds(NH, 8)])
        plsc.subcore_barrier()

        def scan_chunk(t, _):
            pltpu.sync_copy(dst_hbm.at[pl.ds(base_a + t * CH, CH)], dstc)

            def scan(u, _):
                j = t * (CH // VR) + u
                b = ib2d[j // (BR // VR), pl.ds((j % (BR // VR)) * VR, VR)]
                dv = dstc[pl.ds(u * VR, VR)]
                owned = (dv >= c * NH) & (dv < c * NH + NH)
                need = (((b >> i) & 1) == 0) & owned
                nact = jnp.max(need.astype(jnp.int32))

                @pl.when(nact > 0)
                def _():
                    pltpu.sync_copy(src_hbm.at[pl.ds(base_a + j * VR, VR)],
                                    idx16a)
                    idx16b[...] = jnp.where(need, dv - c * NH, NH + (lane & 7))
                    pltpu.sync_copy(xt_hbm.at[idx16a], rows.at[pl.ds(0, VR)])
                    pltpu.sync_copy(rows.at[pl.ds(0, VR)], acc_sh.at[idx16b],
                                    add=True)
                    _zero_f32_2d(rows, VR, F)
                return 0
            lax.fori_loop(0, CH // VR, scan, 0)
            return 0
        lax.fori_loop(0, NCH, scan_chunk, 0)
        plsc.subcore_barrier()
        pltpu.sync_copy(acc_sh.at[pl.ds(crow0, CSL)],
                        c_out.at[i, pl.ds(c * NH + crow0, CSL)])
        plsc.subcore_barrier()
        return 0
    lax.fori_loop(0, D, corr, 0)


_sc_call = pl.kernel(
    _sc_body,
    out_type=[
        jax.ShapeDtypeStruct((NC, NPAD, F), jnp.float32),
        jax.ShapeDtypeStruct((D, NPAD, F), jnp.float32),
    ],
    mesh=plsc.VectorSubcoreMesh(core_axis_name="c", subcore_axis_name="s"),
    scratch_types=[
        pltpu.VMEM_SHARED((D * NPAD,), jnp.float32),
        pltpu.VMEM_SHARED((NPAD,), jnp.int32),
        pltpu.VMEM_SHARED((NPAD, F), jnp.float32),
        pltpu.VMEM((ERWS, BR), jnp.int32),
        pltpu.VMEM((NPAD,), jnp.int32),
        pltpu.VMEM((BC, F), jnp.float32),
        pltpu.VMEM((8, BR), jnp.int32),
        pltpu.VMEM((NSL,), jnp.float32),
        pltpu.VMEM((NSL,), jnp.int32),
        pltpu.VMEM((CH,), jnp.int32),
        pltpu.VMEM((CH,), jnp.int32),
        pltpu.VMEM((CH,), jnp.int32),
        pltpu.VMEM((BR,), jnp.float32),
        pltpu.VMEM((BC,), jnp.int32),
        pltpu.VMEM((BC,), jnp.int32),
        pltpu.VMEM((VR,), jnp.int32),
        pltpu.VMEM((VR,), jnp.int32),
        pltpu.SemaphoreType.DMA,
    ],
    compiler_params=pltpu.CompilerParams(needs_layout_passes=False),
)


def _xt_body(f_ref, n_ref, o_ref):
    o_ref[...] = f_ref[...] * n_ref[...]


_xt_call = pl.pallas_call(
    _xt_body,
    grid=(10,),
    in_specs=[
        pl.BlockSpec((N // 10, F), lambda i: (i, 0)),
        pl.BlockSpec((N // 10, 1), lambda i: (i, 0)),
    ],
    out_specs=pl.BlockSpec((N // 10, F), lambda i: (i, 0)),
    out_shape=jax.ShapeDtypeStruct((N, F), jnp.float32),
)


def _fin_body(s_ref, c_ref, w_ref, n_ref, o_ref):
    si = s_ref[0] + s_ref[1] - c_ref[0]
    m = jnp.dot(si, w_ref[0], preferred_element_type=jnp.float32)
    o_ref[...] = jnp.maximum(m * n_ref[...], 0.0)


BN = 256
_fin_call = pl.pallas_call(
    _fin_body,
    grid=(NPAD // BN, D),
    in_specs=[
        pl.BlockSpec((NC, BN, F), lambda n, i: (0, n, 0)),
        pl.BlockSpec((1, BN, F), lambda n, i: (i, n, 0)),
        pl.BlockSpec((1, F, F), lambda n, i: (i, 0, 0)),
        pl.BlockSpec((BN, 1), lambda n, i: (n, 0)),
    ],
    out_specs=pl.BlockSpec((BN, F), lambda n, i: (n, i)),
    out_shape=jax.ShapeDtypeStruct((NPAD, D * F), jnp.float32),
)


def kernel(feature, edge_index, edge_subgraph_idx, norm, W):
    xt = _xt_call(feature, norm)
    src, dst = edge_index[0], edge_index[1]
    s_part, c_corr = _sc_call(xt, src, dst, edge_subgraph_idx)
    norm_pad = jnp.zeros((NPAD, 1), jnp.float32).at[:N].set(norm)
    out = _fin_call(s_part, c_corr, W, norm_pad)
    return out[:N]
